# baseline (speedup 1.0000x reference)
import functools

import jax
import jax.numpy as jnp
from jax import lax
from jax.experimental import pallas as pl
from jax.experimental.pallas import tpu as pltpu
from jax.experimental.pallas import tpu_sc as plsc

B, S, H = 16, 512, 1024
NUM_TAGS = 64
INTER = 4096
NUM_LABELS = 17
TAGGING_RATE = 0.5
EPS = 1e-12

N = B * S
HP = H // 2

_NC = 2
_NS = 16
_NW = _NC * _NS
_PER_W = N // _NW
_CH = 64
_NCH = _PER_W // _CH


def _sc_gather_body(tab_hbm, ids_hbm, out_hbm, idx_v, buf0, buf1,
                    g0, g1, s0, s1):
    wid = lax.axis_index("s") * _NC + lax.axis_index("c")
    base = wid * _PER_W
    bufs = [buf0, buf1]
    gsems = [g0, g1]
    ssems = [s0, s1]
    gather = [None, None]
    scatter = [None, None]
    pltpu.sync_copy(ids_hbm.at[pl.ds(base, _PER_W)], idx_v)
    for ch in range(_NCH):
        p = ch & 1
        if scatter[p] is not None:
            scatter[p].wait()
            scatter[p] = None
        gather[p] = pltpu.async_copy(
            tab_hbm.at[idx_v.at[pl.ds(ch * _CH, _CH)]], bufs[p], gsems[p])
        q = p ^ 1
        if gather[q] is not None:
            gather[q].wait()
            gather[q] = None
            scatter[q] = pltpu.async_copy(
                bufs[q], out_hbm.at[pl.ds(base + (ch - 1) * _CH, _CH)],
                ssems[q])
    p_last = (_NCH - 1) & 1
    gather[p_last].wait()
    scatter[p_last] = pltpu.async_copy(
        bufs[p_last], out_hbm.at[pl.ds(base + (_NCH - 1) * _CH, _CH)],
        ssems[p_last])
    for s in scatter:
        if s is not None:
            s.wait()


@functools.cache
def _sc_gather():
    return pl.kernel(
        _sc_gather_body,
        out_type=jax.ShapeDtypeStruct((N, HP), jnp.int32),
        mesh=plsc.VectorSubcoreMesh(core_axis_name="c", subcore_axis_name="s"),
        scratch_types=[
            pltpu.VMEM((_PER_W,), jnp.int32),
            pltpu.VMEM((_CH, HP), jnp.int32),
            pltpu.VMEM((_CH, HP), jnp.int32),
            pltpu.SemaphoreType.DMA,
            pltpu.SemaphoreType.DMA,
            pltpu.SemaphoreType.DMA,
            pltpu.SemaphoreType.DMA,
        ],
    )


_TB = 512
_SPLIT = 4


def _ln(x, g, b):
    s1 = jnp.sum(x, axis=-1, keepdims=True)
    s2 = jnp.sum(x * x, axis=-1, keepdims=True)
    mu = s1 * (1.0 / H)
    var = s2 * (1.0 / H) - mu * mu
    return (x - mu) * lax.rsqrt(var + EPS) * g + b


def _tc_body(word, tagged, g1, beta1, W1, b1, W2, b2, Wg, csum, bq,
             out):
    cb = _TB // _SPLIT
    rs_ = [pl.ds(c * cb, cb) for c in range(_SPLIT)]

    xns = []
    for r in rs_:
        t = tagged[r, :]
        tlo = lax.bitcast_convert_type(lax.shift_left(t, 16), jnp.float32)
        thi = lax.bitcast_convert_type(
            jnp.bitwise_and(t, jnp.int32(-65536)), jnp.float32)
        tag = jnp.concatenate([tlo, thi], axis=-1)
        x = word[r, :] + TAGGING_RATE * tag
        xns.append(_ln(x, g1[...], beta1[...]))

    hs = []
    for xn in xns:
        h = jnp.dot(xn.astype(jnp.bfloat16), W1[...],
                    preferred_element_type=jnp.float32) + b1[...]
        hs.append(jnp.maximum(h, 0.0).astype(jnp.bfloat16))

    ys = []
    for xn, h in zip(xns, hs):
        ys.append(jnp.dot(h, W2[...], preferred_element_type=jnp.float32)
                  + b2[...] + xn)

    for r, y in zip(rs_, ys):
        s1 = jnp.sum(y, axis=-1, keepdims=True)
        s2 = jnp.sum(y * y, axis=-1, keepdims=True)
        mu = s1 * (1.0 / H)
        var = s2 * (1.0 / H) - mu * mu
        rstd = lax.rsqrt(var + EPS)
        z = jnp.dot(y, Wg[...], preferred_element_type=jnp.float32)
        out[r, :] = rstd * (z - mu * csum[...]) + bq[...]


def _tc_call(word2d, tagged2d, g1, beta1, W1, b1, W2, b2, Wg, csum, bq):
    nb = N // _TB
    tok = lambda i: (i, 0)
    const = lambda i: (0, 0)
    vec = pl.BlockSpec((1, H), const)
    return pl.pallas_call(
        _tc_body,
        grid=(nb,),
        in_specs=[
            pl.BlockSpec((_TB, H), tok),
            pl.BlockSpec((_TB, HP), tok),
            vec, vec,
            pl.BlockSpec((H, INTER), const),
            pl.BlockSpec((1, INTER), const),
            pl.BlockSpec((INTER, H), const),
            pl.BlockSpec((1, H), const),
            pl.BlockSpec((H, NUM_LABELS), const),
            pl.BlockSpec((1, NUM_LABELS), const),
            pl.BlockSpec((1, NUM_LABELS), const),
        ],
        out_specs=pl.BlockSpec((_TB, NUM_LABELS), tok),
        out_shape=jax.ShapeDtypeStruct((N, NUM_LABELS), jnp.float32),
        compiler_params=pltpu.CompilerParams(
            dimension_semantics=("arbitrary",),
        ),
    )(word2d, tagged2d, g1, beta1, W1, b1, W2, b2, Wg, csum, bq)


def kernel(word_embedding, tag_to_spans, tag_embedding, att_gamma, att_beta,
           W1, b1, W2, b2, ff_gamma, ff_beta, Wout, bout):
    ids = tag_to_spans.reshape(N)
    word2d = word_embedding.reshape(N, H)
    tab_bf = tag_embedding.astype(jnp.bfloat16)
    tab_packed = lax.bitcast_convert_type(
        jnp.stack([tab_bf[:, :HP], tab_bf[:, HP:]], axis=-1), jnp.int32)

    Wg = ff_gamma[:, None] * Wout
    csum = jnp.sum(Wg, axis=0).reshape(1, NUM_LABELS)
    bq = (ff_beta @ Wout + bout).reshape(1, NUM_LABELS)

    tagged = _sc_gather()(tab_packed, ids)
    out = _tc_call(
        word2d, tagged,
        att_gamma.reshape(1, H), att_beta.reshape(1, H),
        W1.astype(jnp.bfloat16), b1.reshape(1, INTER),
        W2.astype(jnp.bfloat16), b2.reshape(1, H),
        Wg, csum, bq,
    )
    return out.reshape(B, S, NUM_LABELS)


# speedup vs baseline: 1.0245x; 1.0245x over previous
import functools

import jax
import jax.numpy as jnp
from jax import lax
from jax.experimental import pallas as pl
from jax.experimental.pallas import tpu as pltpu
from jax.experimental.pallas import tpu_sc as plsc

B, S, H = 16, 512, 1024
NUM_TAGS = 64
INTER = 4096
NUM_LABELS = 17
TAGGING_RATE = 0.5
EPS = 1e-12

N = B * S
HP = H // 2

_NC = 2
_NS = 16
_NW = _NC * _NS
_PER_W = N // _NW
_CH = 64
_NCH = _PER_W // _CH


def _sc_gather_body(tab_hbm, ids_hbm, out_hbm, idx_v, buf0, buf1,
                    g0, g1, s0, s1):
    wid = lax.axis_index("s") * _NC + lax.axis_index("c")
    base = wid * _PER_W
    bufs = [buf0, buf1]
    gsems = [g0, g1]
    ssems = [s0, s1]
    gather = [None, None]
    scatter = [None, None]
    pltpu.sync_copy(ids_hbm.at[pl.ds(base, _PER_W)], idx_v)
    for ch in range(_NCH):
        p = ch & 1
        if scatter[p] is not None:
            scatter[p].wait()
            scatter[p] = None
        gather[p] = pltpu.async_copy(
            tab_hbm.at[idx_v.at[pl.ds(ch * _CH, _CH)]], bufs[p], gsems[p])
        q = p ^ 1
        if gather[q] is not None:
            gather[q].wait()
            gather[q] = None
            scatter[q] = pltpu.async_copy(
                bufs[q], out_hbm.at[pl.ds(base + (ch - 1) * _CH, _CH)],
                ssems[q])
    p_last = (_NCH - 1) & 1
    gather[p_last].wait()
    scatter[p_last] = pltpu.async_copy(
        bufs[p_last], out_hbm.at[pl.ds(base + (_NCH - 1) * _CH, _CH)],
        ssems[p_last])
    for s in scatter:
        if s is not None:
            s.wait()


@functools.cache
def _sc_gather():
    return pl.kernel(
        _sc_gather_body,
        out_type=jax.ShapeDtypeStruct((N, HP), jnp.int32),
        mesh=plsc.VectorSubcoreMesh(core_axis_name="c", subcore_axis_name="s"),
        scratch_types=[
            pltpu.VMEM((_PER_W,), jnp.int32),
            pltpu.VMEM((_CH, HP), jnp.int32),
            pltpu.VMEM((_CH, HP), jnp.int32),
            pltpu.SemaphoreType.DMA,
            pltpu.SemaphoreType.DMA,
            pltpu.SemaphoreType.DMA,
            pltpu.SemaphoreType.DMA,
        ],
    )


_TB = 512
_SPLIT = 4


def _ln(x, g, b):
    s1 = jnp.sum(x, axis=-1, keepdims=True)
    s2 = jnp.sum(x * x, axis=-1, keepdims=True)
    mu = s1 * (1.0 / H)
    var = s2 * (1.0 / H) - mu * mu
    return (x - mu) * lax.rsqrt(var + EPS) * g + b


def _tc_body(word, tagged, g1, beta1, W1, b1, W2, b2, Wg, csum, bq,
             out):
    cb = _TB // _SPLIT
    rs_ = [pl.ds(c * cb, cb) for c in range(_SPLIT)]

    xns = []
    for r in rs_:
        t = tagged[r, :]
        tlo = lax.bitcast_convert_type(lax.shift_left(t, 16), jnp.float32)
        thi = lax.bitcast_convert_type(
            jnp.bitwise_and(t, jnp.int32(-65536)), jnp.float32)
        tag = jnp.concatenate([tlo, thi], axis=-1)
        x = word[r, :] + TAGGING_RATE * tag
        xns.append(_ln(x, g1[...], beta1[...]))

    hs = []
    for xn in xns:
        h = jnp.dot(xn, W1[...], preferred_element_type=jnp.float32) + b1[...]
        hs.append(jnp.maximum(h, 0.0))

    ys = []
    for xn, h in zip(xns, hs):
        ys.append(jnp.dot(h, W2[...], preferred_element_type=jnp.float32)
                  + b2[...] + xn)

    for r, y in zip(rs_, ys):
        s1 = jnp.sum(y, axis=-1, keepdims=True)
        s2 = jnp.sum(y * y, axis=-1, keepdims=True)
        mu = s1 * (1.0 / H)
        var = s2 * (1.0 / H) - mu * mu
        rstd = lax.rsqrt(var + EPS)
        z = jnp.dot(y, Wg[...], preferred_element_type=jnp.float32)
        out[r, :] = rstd * (z - mu * csum[...]) + bq[...]


def _tc_call(word2d, tagged2d, g1, beta1, W1, b1, W2, b2, Wg, csum, bq):
    nb = N // _TB
    tok = lambda i: (i, 0)
    const = lambda i: (0, 0)
    vec = pl.BlockSpec((1, H), const)
    return pl.pallas_call(
        _tc_body,
        grid=(nb,),
        in_specs=[
            pl.BlockSpec((_TB, H), tok),
            pl.BlockSpec((_TB, HP), tok),
            vec, vec,
            pl.BlockSpec((H, INTER), const),
            pl.BlockSpec((1, INTER), const),
            pl.BlockSpec((INTER, H), const),
            pl.BlockSpec((1, H), const),
            pl.BlockSpec((H, NUM_LABELS), const),
            pl.BlockSpec((1, NUM_LABELS), const),
            pl.BlockSpec((1, NUM_LABELS), const),
        ],
        out_specs=pl.BlockSpec((_TB, NUM_LABELS), tok),
        out_shape=jax.ShapeDtypeStruct((N, NUM_LABELS), jnp.float32),
        compiler_params=pltpu.CompilerParams(
            dimension_semantics=("arbitrary",),
        ),
    )(word2d, tagged2d, g1, beta1, W1, b1, W2, b2, Wg, csum, bq)


def kernel(word_embedding, tag_to_spans, tag_embedding, att_gamma, att_beta,
           W1, b1, W2, b2, ff_gamma, ff_beta, Wout, bout):
    ids = tag_to_spans.reshape(N)
    word2d = word_embedding.reshape(N, H)
    tab_bf = tag_embedding.astype(jnp.bfloat16)
    tab_packed = lax.bitcast_convert_type(
        jnp.stack([tab_bf[:, :HP], tab_bf[:, HP:]], axis=-1), jnp.int32)

    Wg = ff_gamma[:, None] * Wout
    csum = jnp.sum(Wg, axis=0).reshape(1, NUM_LABELS)
    bq = (ff_beta @ Wout + bout).reshape(1, NUM_LABELS)

    tagged = _sc_gather()(tab_packed, ids)
    out = _tc_call(
        word2d, tagged,
        att_gamma.reshape(1, H), att_beta.reshape(1, H),
        W1, b1.reshape(1, INTER),
        W2, b2.reshape(1, H),
        Wg, csum, bq,
    )
    return out.reshape(B, S, NUM_LABELS)
